# Initial kernel scaffold; baseline (speedup 1.0000x reference)
import jax
import jax.numpy as jnp
from jax.experimental import pallas as pl


def kernel(indices, table):
    raise NotImplementedError("write your pallas kernel here")


# speedup vs baseline: 24.5150x; 24.5150x over previous
import functools

import jax
import jax.numpy as jnp
from jax import lax
from jax.experimental import pallas as pl
from jax.experimental.pallas import tpu as pltpu
from jax.experimental.pallas import tpu_sc as plsc

B = 4096
L = 200
D = 32
N = B * L

IDX_W = 128
K = 8
CH = K * IDX_W

_info = plsc.get_sparse_core_info()
NC, NS = _info.num_cores, _info.num_subcores
NW = NC * NS
ROWS_PER_W = N // NW
CHUNKS = ROWS_PER_W // CH
assert ROWS_PER_W % CH == 0


def _emb_gather_body(idx_hbm, table_hbm, out_hbm, idx_v, rows_v, sem):
    wid = lax.axis_index("s") * NC + lax.axis_index("c")
    base0 = wid * ROWS_PER_W

    def chunk(c, _):
        base = pl.multiple_of(base0 + c * CH, CH)
        pltpu.sync_copy(idx_hbm.at[pl.ds(pl.multiple_of(base // IDX_W, K), K)], idx_v)
        cps = [
            pltpu.async_copy(
                table_hbm.at[idx_v.at[j]],
                rows_v.at[pl.ds(j * IDX_W, IDX_W)],
                sem,
            )
            for j in range(K)
        ]
        for cp in cps:
            cp.wait()
        pltpu.sync_copy(rows_v, out_hbm.at[pl.ds(base, CH)])
        return 0

    lax.fori_loop(0, CHUNKS, chunk, 0)


@functools.partial(jax.jit, donate_argnums=())
def _emb_gather(idx2d, table):
    run = functools.partial(
        pl.kernel,
        out_type=jax.ShapeDtypeStruct((N, D), jnp.float32),
        mesh=plsc.VectorSubcoreMesh(core_axis_name="c", subcore_axis_name="s"),
        scratch_types=[
            pltpu.VMEM((K, IDX_W), jnp.int32),
            pltpu.VMEM((CH, D), jnp.float32),
            pltpu.SemaphoreType.DMA,
        ],
        compiler_params=pltpu.CompilerParams(use_tc_tiling_on_sc=False),
    )(_emb_gather_body)
    return run(idx2d, table)


def kernel(indices, table):
    idx2d = jnp.transpose(indices).reshape(N // IDX_W, IDX_W).astype(jnp.int32)
    out = _emb_gather(idx2d, table)
    return out.reshape(L, B, D)


# speedup vs baseline: 25.5385x; 1.0417x over previous
import functools

import jax
import jax.numpy as jnp
from jax import lax
from jax.experimental import pallas as pl
from jax.experimental.pallas import tpu as pltpu
from jax.experimental.pallas import tpu_sc as plsc

B = 4096
L = 200
D = 32
N = B * L

IDX_W = 128
K = 10
CH = K * IDX_W

_info = plsc.get_sparse_core_info()
NC, NS = _info.num_cores, _info.num_subcores
NW = NC * NS
ROWS_PER_W = N // NW
CHUNKS = ROWS_PER_W // CH
assert ROWS_PER_W % CH == 0 and CHUNKS % 2 == 0


def _emb_gather_body(idx_hbm, table_hbm, out_hbm,
                     idx0, idx1, rows0, rows1,
                     isem0, isem1, gsem0, gsem1, ssem0, ssem1):
    wid = lax.axis_index("s") * NC + lax.axis_index("c")
    base0 = wid * ROWS_PER_W
    bufs = ((idx0, isem0, rows0, gsem0, ssem0),
            (idx1, isem1, rows1, gsem1, ssem1))

    def idx_rows(c):
        return pl.multiple_of((base0 + c * CH) // IDX_W, K)

    def start_idx(c, p):
        idx_v, isem = bufs[p][0], bufs[p][1]
        pltpu.make_async_copy(
            idx_hbm.at[pl.ds(idx_rows(c), K)], idx_v, isem).start()

    def do_chunk(c, p, first):
        idx_v, isem, rows_v, gsem, ssem = bufs[p]
        base = pl.multiple_of(base0 + c * CH, CH)
        pltpu.make_async_copy(
            idx_hbm.at[pl.ds(idx_rows(0), K)], idx_v, isem).wait()
        if not first:
            pltpu.make_async_copy(
                rows_v, out_hbm.at[pl.ds(base, CH)], ssem).wait()
        gathers = [
            pltpu.make_async_copy(
                table_hbm.at[idx_v.at[j]],
                rows_v.at[pl.ds(j * IDX_W, IDX_W)],
                gsem,
            )
            for j in range(K)
        ]
        for g in gathers:
            g.start()
        for g in gathers:
            g.wait()
        cn = jnp.minimum(c + 2, CHUNKS - 1)
        start_idx(cn, p)
        pltpu.make_async_copy(
            rows_v, out_hbm.at[pl.ds(base, CH)], ssem).start()

    start_idx(0, 0)
    start_idx(1, 1)
    do_chunk(0, 0, True)
    do_chunk(1, 1, True)

    def pair(cp, _):
        c = cp * 2
        do_chunk(c, 0, False)
        do_chunk(c + 1, 1, False)
        return 0

    lax.fori_loop(1, CHUNKS // 2, pair, 0)

    for p in (0, 1):
        idx_v, isem, rows_v, _, ssem = bufs[p]
        pltpu.make_async_copy(
            idx_hbm.at[pl.ds(idx_rows(0), K)], idx_v, isem).wait()
        pltpu.make_async_copy(
            rows_v, out_hbm.at[pl.ds(pl.multiple_of(base0, CH), CH)], ssem).wait()


@jax.jit
def _emb_gather(idx2d, table):
    run = functools.partial(
        pl.kernel,
        out_type=jax.ShapeDtypeStruct((N, D), jnp.float32),
        mesh=plsc.VectorSubcoreMesh(core_axis_name="c", subcore_axis_name="s"),
        scratch_types=[
            pltpu.VMEM((K, IDX_W), jnp.int32),
            pltpu.VMEM((K, IDX_W), jnp.int32),
            pltpu.VMEM((CH, D), jnp.float32),
            pltpu.VMEM((CH, D), jnp.float32),
            pltpu.SemaphoreType.DMA,
            pltpu.SemaphoreType.DMA,
            pltpu.SemaphoreType.DMA,
            pltpu.SemaphoreType.DMA,
            pltpu.SemaphoreType.DMA,
            pltpu.SemaphoreType.DMA,
        ],
        compiler_params=pltpu.CompilerParams(use_tc_tiling_on_sc=False),
    )(_emb_gather_body)
    return run(idx2d, table)


def kernel(indices, table):
    idx2d = jnp.transpose(indices).reshape(N // IDX_W, IDX_W).astype(jnp.int32)
    out = _emb_gather(idx2d, table)
    return out.reshape(L, B, D)
